# baseline (speedup 1.0000x reference)
import functools

import jax
import jax.numpy as jnp
import numpy as np
from jax import lax
from jax.experimental import pallas as pl
from jax.experimental.pallas import tpu as pltpu
from jax.experimental.pallas import tpu_sc as plsc

N, B, K, DF, NH, NHEAD = 100000, 512, 20, 128, 128, 2
DM = 2 * NH
DK = DM // NHEAD
M2 = B * K
KP = 24


def _sc_gather(table, idx):
    NW, n_ch, CH = idx.shape
    D = table.shape[1]
    per_w = n_ch * CH
    T = NW * per_w
    info = plsc.get_sparse_core_info()
    NC = info.num_cores
    mesh = plsc.VectorSubcoreMesh(core_axis_name="c", subcore_axis_name="s")
    idx_flat = idx.reshape(T)

    @functools.partial(
        pl.kernel, mesh=mesh,
        out_type=jax.ShapeDtypeStruct((T, D), jnp.float32),
        scratch_types=[
            pltpu.VMEM((CH,), jnp.int32),
            pltpu.VMEM((CH, D), jnp.float32),
            pltpu.SemaphoreType.DMA,
        ],
    )
    def k(table_hbm, idx_hbm, out_hbm, idx_v, rows_v, sem):
        wid = lax.axis_index("s") * NC + lax.axis_index("c")
        base0 = wid * per_w

        def body(c, _):
            base = base0 + c * CH
            pltpu.sync_copy(idx_hbm.at[pl.ds(base, CH)], idx_v)
            pltpu.async_copy(table_hbm.at[idx_v], rows_v, sem).wait()
            pltpu.sync_copy(rows_v, out_hbm.at[pl.ds(base, CH)])
            return _

        lax.fori_loop(0, n_ch, body, None)

    return k(table, idx_flat)


def _attn_math(src_conv, seq, dt3, nidx3, freq3, phase3,
               Wq, Wk, Wv, f1w, f1b, f2w, f2b, W_lin=None, b_lin=None):
    BM = src_conv.shape[0]
    tenc0 = jnp.cos(phase3[0])
    q = jnp.concatenate(
        [src_conv, jnp.broadcast_to(tenc0, (BM, NH))], axis=1)
    Q = jnp.dot(q, Wq, preferred_element_type=jnp.float32)
    tenc3 = jnp.cos(dt3 * freq3 + phase3)
    mask3 = nidx3 == 0
    scale = 1.0 / np.sqrt(DK)
    outs = []
    for h in range(NHEAD):
        Qh = Q[:, h * DK:(h + 1) * DK]
        Wkh = Wk[:, h * DK:(h + 1) * DK]
        Qt = lax.dot_general(Qh, Wkh, (((1,), (1,)), ((), ())),
                             preferred_element_type=jnp.float32)
        Qt_f, Qt_t = Qt[:, :NH], Qt[:, NH:]
        if W_lin is not None:
            Qr = lax.dot_general(Qt_f, W_lin, (((1,), (1,)), ((), ())),
                                 preferred_element_type=jnp.float32)
        else:
            Qr = Qt_f
        s3 = (jnp.sum(seq * Qr[:, None, :], axis=2, keepdims=True)
              + jnp.sum(tenc3 * Qt_t[:, None, :], axis=2,
                        keepdims=True))
        s3 = jnp.where(mask3, -1e10, s3 * scale)
        kpad = lax.broadcasted_iota(jnp.int32, (BM, KP, 1), 1) >= K
        s3 = jnp.where(kpad, -jnp.inf, s3)
        smax = jnp.max(s3, axis=1, keepdims=True)
        e = jnp.exp(s3 - smax)
        a3 = e / jnp.sum(e, axis=1, keepdims=True)
        cr = jnp.sum(seq * a3, axis=1)
        ct = jnp.sum(tenc3 * a3, axis=1)
        Wvh = Wv[:, h * DK:(h + 1) * DK]
        if W_lin is not None:
            cr = jnp.dot(cr, W_lin,
                         preferred_element_type=jnp.float32) + b_lin
        oh = (jnp.dot(cr, Wvh[:NH, :], preferred_element_type=jnp.float32)
              + jnp.dot(ct, Wvh[NH:, :], preferred_element_type=jnp.float32))
        outs.append(oh)
    o = jnp.concatenate(outs, axis=1)
    x = jnp.concatenate([o, src_conv], axis=1)
    h1 = jax.nn.relu(jnp.dot(x, f1w, preferred_element_type=jnp.float32)
                     + f1b)
    return jnp.dot(h1, f2w, preferred_element_type=jnp.float32) + f2b


def _big_body(src_raw_ref, seq_ref, st_ref, nt_ref, nidx_ref,
              wlin_ref, blin_ref, freq_ref, phase_ref,
              wq_ref, wk_ref, wv_ref, f1w_ref, f1b_ref, f2w_ref, f2b_ref,
              out_ref):
    src_conv = (jnp.dot(src_raw_ref[...], wlin_ref[...],
                        preferred_element_type=jnp.float32)
                + blin_ref[...])
    dt3 = (st_ref[...] - nt_ref[...])[:, :, None]
    out_ref[...] = _attn_math(
        src_conv, seq_ref[...], dt3, nidx_ref[...][:, :, None],
        freq_ref[...], phase_ref[...],
        wq_ref[...], wk_ref[...], wv_ref[...],
        f1w_ref[...], f1b_ref[...], f2w_ref[...], f2b_ref[...],
        W_lin=wlin_ref[...], b_lin=blin_ref[...])


def _attn_big(src_raw, seq, src_t, nt, nidx, W_lin, b_lin2, freq3, phase3,
              Wq, Wk, Wv, f1w, f1b2, f2w, f2b2, bm):
    m = src_raw.shape[0]
    grid = (m // bm,)
    row = lambda i: (i, 0)
    row3 = lambda i: (i, 0, 0)
    fixed = lambda i: (0, 0)
    fixed3 = lambda i: (0, 0, 0)
    return pl.pallas_call(
        _big_body,
        grid=grid,
        in_specs=[
            pl.BlockSpec((bm, NH), row),
            pl.BlockSpec((bm, KP, NH), row3),
            pl.BlockSpec((bm, 1), row),
            pl.BlockSpec((bm, KP), row),
            pl.BlockSpec((bm, KP), row),
            pl.BlockSpec((DF, NH), fixed),
            pl.BlockSpec((1, NH), fixed),
            pl.BlockSpec((1, 1, NH), fixed3),
            pl.BlockSpec((1, 1, NH), fixed3),
            pl.BlockSpec((DM, DM), fixed),
            pl.BlockSpec((DM, DM), fixed),
            pl.BlockSpec((DM, DM), fixed),
            pl.BlockSpec((DM + NH, NH), fixed),
            pl.BlockSpec((1, NH), fixed),
            pl.BlockSpec((NH, NH), fixed),
            pl.BlockSpec((1, NH), fixed),
        ],
        out_specs=pl.BlockSpec((bm, NH), row),
        out_shape=jax.ShapeDtypeStruct((m, NH), jnp.float32),
    )(src_raw, seq, src_t, nt, nidx, W_lin, b_lin2, freq3, phase3,
      Wq, Wk, Wv, f1w, f1b2, f2w, f2b2)


def _small_body(src_raw_ref, seq1_ref, seq2_ref, ct_ref, nt_ref, nidx_ref,
                wlin_ref, blin_ref, freq_ref, phase_ref,
                wq0_ref, wk0_ref, wv0_ref, f1w0_ref, f1b0_ref, f2w0_ref,
                f2b0_ref,
                wq1_ref, wk1_ref, wv1_ref, f1w1_ref, f1b1_ref, f2w1_ref,
                f2b1_ref,
                out_ref):
    src_conv = (jnp.dot(src_raw_ref[...], wlin_ref[...],
                        preferred_element_type=jnp.float32)
                + blin_ref[...])
    dt3 = (ct_ref[...] - nt_ref[...])[:, :, None]
    freq3, phase3 = freq_ref[...], phase_ref[...]
    nidx3 = nidx_ref[...][:, :, None]
    src_l1 = _attn_math(
        src_conv, seq1_ref[...], dt3, nidx3, freq3, phase3,
        wq0_ref[...], wk0_ref[...], wv0_ref[...],
        f1w0_ref[...], f1b0_ref[...], f2w0_ref[...], f2b0_ref[...],
        W_lin=wlin_ref[...], b_lin=blin_ref[...])
    out_ref[...] = _attn_math(
        src_l1, seq2_ref[...], dt3, nidx3, freq3, phase3,
        wq1_ref[...], wk1_ref[...], wv1_ref[...],
        f1w1_ref[...], f1b1_ref[...], f2w1_ref[...], f2b1_ref[...])


def _attn_small(src_raw, seq1, seq2, cut_t, nt, nidx,
                W_lin, b_lin2, freq3, phase3, w0, w1, bm):
    m = src_raw.shape[0]
    grid = (m // bm,)
    row = lambda i: (i, 0)
    row3 = lambda i: (i, 0, 0)
    fixed = lambda i: (0, 0)
    fixed3 = lambda i: (0, 0, 0)
    wspecs = [
        pl.BlockSpec((DM, DM), fixed),
        pl.BlockSpec((DM, DM), fixed),
        pl.BlockSpec((DM, DM), fixed),
        pl.BlockSpec((DM + NH, NH), fixed),
        pl.BlockSpec((1, NH), fixed),
        pl.BlockSpec((NH, NH), fixed),
        pl.BlockSpec((1, NH), fixed),
    ]
    return pl.pallas_call(
        _small_body,
        grid=grid,
        in_specs=[
            pl.BlockSpec((bm, NH), row),
            pl.BlockSpec((bm, KP, NH), row3),
            pl.BlockSpec((bm, KP, NH), row3),
            pl.BlockSpec((bm, 1), row),
            pl.BlockSpec((bm, KP), row),
            pl.BlockSpec((bm, KP), row),
            pl.BlockSpec((DF, NH), fixed),
            pl.BlockSpec((1, NH), fixed),
            pl.BlockSpec((1, 1, NH), fixed3),
            pl.BlockSpec((1, 1, NH), fixed3),
        ] + wspecs + wspecs,
        out_specs=pl.BlockSpec((bm, NH), row),
        out_shape=jax.ShapeDtypeStruct((m, NH), jnp.float32),
    )(src_raw, seq1, seq2, cut_t, nt, nidx, W_lin, b_lin2, freq3, phase3,
      *w0, *w1)


def kernel(node_feat, src_idx, cut_time, ngh_idx_l1, ngh_t_l1, ngh_idx_l2,
           ngh_t_l2, W_lin, b_lin, freq, phase, a0_Wq, a0_Wk, a0_Wv,
           a0_fc1_w, a0_fc1_b, a0_fc2_w, a0_fc2_b, a1_Wq, a1_Wk, a1_Wv,
           a1_fc1_w, a1_fc1_b, a1_fc2_w, a1_fc2_b):
    idx24_l2 = jnp.pad(ngh_idx_l2.astype(jnp.int32).reshape(M2, K),
                       ((0, 0), (0, KP - K)))
    idx24_l1 = jnp.pad(ngh_idx_l1.astype(jnp.int32),
                       ((0, 0), (0, KP - K)))
    nt24_l2 = jnp.pad(ngh_t_l2, ((0, 0), (0, KP - K)))
    nt24_l1 = jnp.pad(ngh_t_l1, ((0, 0), (0, KP - K)))

    n_l2p = M2 * KP
    n_l1p = B * KP
    n_l1 = M2
    n_src = B
    total = n_l2p + n_l1p + n_l1 + n_src
    pad_to = 32 * 256
    t_pad = ((total + pad_to - 1) // pad_to) * pad_to
    all_idx = jnp.concatenate([
        idx24_l2.reshape(-1),
        idx24_l1.reshape(-1),
        ngh_idx_l1.reshape(-1).astype(jnp.int32),
        src_idx.astype(jnp.int32),
        jnp.zeros((t_pad - total,), jnp.int32),
    ]).reshape(32, -1, 128)
    gathered = _sc_gather(node_feat, all_idx)
    o1 = n_l2p
    o2 = o1 + n_l1p
    o3 = o2 + n_l1
    g_l2 = gathered[:o1].reshape(M2, KP, NH)
    g_seq1 = gathered[o1:o2].reshape(B, KP, NH)
    g_l1 = gathered[o2:o3]
    g_src = gathered[o3:o3 + n_src]

    b_lin2 = b_lin.reshape(1, NH)
    freq3 = freq.reshape(1, 1, NH)
    phase3 = phase.reshape(1, 1, NH)
    f1b0 = a0_fc1_b.reshape(1, NH)
    f2b0 = a0_fc2_b.reshape(1, NH)
    f1b1 = a1_fc1_b.reshape(1, NH)
    f2b1 = a1_fc2_b.reshape(1, NH)

    ngh_l1 = _attn_big(
        g_l1, g_l2, ngh_t_l1.reshape(M2, 1), nt24_l2, idx24_l2,
        W_lin, b_lin2, freq3, phase3,
        a0_Wq, a0_Wk, a0_Wv, a0_fc1_w, f1b0, a0_fc2_w, f2b0, bm=256)

    seq2 = jnp.pad(ngh_l1.reshape(B, K, NH), ((0, 0), (0, KP - K), (0, 0)))
    w0 = (a0_Wq, a0_Wk, a0_Wv, a0_fc1_w, f1b0, a0_fc2_w, f2b0)
    w1 = (a1_Wq, a1_Wk, a1_Wv, a1_fc1_w, f1b1, a1_fc2_w, f2b1)
    out = _attn_small(
        g_src, g_seq1, seq2,
        cut_time.reshape(B, 1), nt24_l1, idx24_l1,
        W_lin, b_lin2, freq3, phase3, w0, w1, bm=128)
    return out


# speedup vs baseline: 2.4795x; 2.4795x over previous
import functools

import jax
import jax.numpy as jnp
import numpy as np
from jax import lax
from jax.experimental import pallas as pl
from jax.experimental.pallas import tpu as pltpu
from jax.experimental.pallas import tpu_sc as plsc

N, B, K, DF, NH, NHEAD = 100000, 512, 20, 128, 128, 2
DM = 2 * NH
DK = DM // NHEAD
M2 = B * K
KP = 24


def _sc_gather(table, idx):
    NW, n_ch, CH = idx.shape
    D = table.shape[1]
    per_w = n_ch * CH
    T = NW * per_w
    info = plsc.get_sparse_core_info()
    NC = info.num_cores
    mesh = plsc.VectorSubcoreMesh(core_axis_name="c", subcore_axis_name="s")
    idx_flat = idx.reshape(T)

    @functools.partial(
        pl.kernel, mesh=mesh,
        out_type=jax.ShapeDtypeStruct((T, D), jnp.float32),
        scratch_types=[
            pltpu.VMEM((CH,), jnp.int32),
            pltpu.VMEM((CH, D), jnp.float32),
            pltpu.SemaphoreType.DMA,
        ],
    )
    def k(table_hbm, idx_hbm, out_hbm, idx_v, rows_v, sem):
        wid = lax.axis_index("s") * NC + lax.axis_index("c")
        base0 = wid * per_w

        def body(c, _):
            base = base0 + c * CH
            pltpu.sync_copy(idx_hbm.at[pl.ds(base, CH)], idx_v)
            pltpu.async_copy(table_hbm.at[idx_v], rows_v, sem).wait()
            pltpu.sync_copy(rows_v, out_hbm.at[pl.ds(base, CH)])
            return _

        lax.fori_loop(0, n_ch, body, None)

    return k(table, idx_flat)


def _attn_math(src_conv, seq, dt3, nidx3, freq3, phase3,
               Wq, Wk, Wv, f1w, f1b, f2w, f2b, W_lin=None, b_lin=None):
    BM = src_conv.shape[0]
    tenc0 = jnp.cos(phase3[0])
    q = jnp.concatenate(
        [src_conv, jnp.broadcast_to(tenc0, (BM, NH))], axis=1)
    Q = jnp.dot(q, Wq, preferred_element_type=jnp.float32)
    tenc3 = jnp.cos(dt3 * freq3 + phase3)
    mask3 = nidx3 == 0
    scale = 1.0 / np.sqrt(DK)
    outs = []
    for h in range(NHEAD):
        Qh = Q[:, h * DK:(h + 1) * DK]
        Wkh = Wk[:, h * DK:(h + 1) * DK]
        Qt = lax.dot_general(Qh, Wkh, (((1,), (1,)), ((), ())),
                             preferred_element_type=jnp.float32)
        Qt_f, Qt_t = Qt[:, :NH], Qt[:, NH:]
        if W_lin is not None:
            Qr = lax.dot_general(Qt_f, W_lin, (((1,), (1,)), ((), ())),
                                 preferred_element_type=jnp.float32)
        else:
            Qr = Qt_f
        s3 = (jnp.sum(seq * Qr[:, None, :], axis=2, keepdims=True)
              + jnp.sum(tenc3 * Qt_t[:, None, :], axis=2,
                        keepdims=True))
        s3 = jnp.where(mask3, -1e10, s3 * scale)
        kpad = lax.broadcasted_iota(jnp.int32, (BM, KP, 1), 1) >= K
        s3 = jnp.where(kpad, -jnp.inf, s3)
        smax = jnp.max(s3, axis=1, keepdims=True)
        e = jnp.exp(s3 - smax)
        a3 = e / jnp.sum(e, axis=1, keepdims=True)
        cr = jnp.sum(seq * a3, axis=1)
        ct = jnp.sum(tenc3 * a3, axis=1)
        Wvh = Wv[:, h * DK:(h + 1) * DK]
        if W_lin is not None:
            cr = jnp.dot(cr, W_lin,
                         preferred_element_type=jnp.float32) + b_lin
        oh = (jnp.dot(cr, Wvh[:NH, :], preferred_element_type=jnp.float32)
              + jnp.dot(ct, Wvh[NH:, :], preferred_element_type=jnp.float32))
        outs.append(oh)
    o = jnp.concatenate(outs, axis=1)
    x = jnp.concatenate([o, src_conv], axis=1)
    h1 = jax.nn.relu(jnp.dot(x, f1w, preferred_element_type=jnp.float32)
                     + f1b)
    return jnp.dot(h1, f2w, preferred_element_type=jnp.float32) + f2b


def _big_body(src_raw_ref, seq_ref, st_ref, nt_ref, nidx_ref,
              wlin_ref, blin_ref, freq_ref, phase_ref,
              wq_ref, wk_ref, wv_ref, f1w_ref, f1b_ref, f2w_ref, f2b_ref,
              out_ref):
    src_conv = (jnp.dot(src_raw_ref[...], wlin_ref[...],
                        preferred_element_type=jnp.float32)
                + blin_ref[...])
    dt3 = (st_ref[...] - nt_ref[...])[:, :, None]
    out_ref[...] = _attn_math(
        src_conv, seq_ref[...], dt3, nidx_ref[...][:, :, None],
        freq_ref[...], phase_ref[...],
        wq_ref[...], wk_ref[...], wv_ref[...],
        f1w_ref[...], f1b_ref[...], f2w_ref[...], f2b_ref[...],
        W_lin=wlin_ref[...], b_lin=blin_ref[...])


def _attn_big(src_raw, seq, src_t, nt, nidx, W_lin, b_lin2, freq3, phase3,
              Wq, Wk, Wv, f1w, f1b2, f2w, f2b2, bm):
    m = src_raw.shape[0]
    grid = (m // bm,)
    row = lambda i: (i, 0)
    row3 = lambda i: (i, 0, 0)
    fixed = lambda i: (0, 0)
    fixed3 = lambda i: (0, 0, 0)
    return pl.pallas_call(
        _big_body,
        grid=grid,
        in_specs=[
            pl.BlockSpec((bm, NH), row),
            pl.BlockSpec((bm, KP, NH), row3),
            pl.BlockSpec((bm, 1), row),
            pl.BlockSpec((bm, KP), row),
            pl.BlockSpec((bm, KP), row),
            pl.BlockSpec((DF, NH), fixed),
            pl.BlockSpec((1, NH), fixed),
            pl.BlockSpec((1, 1, NH), fixed3),
            pl.BlockSpec((1, 1, NH), fixed3),
            pl.BlockSpec((DM, DM), fixed),
            pl.BlockSpec((DM, DM), fixed),
            pl.BlockSpec((DM, DM), fixed),
            pl.BlockSpec((DM + NH, NH), fixed),
            pl.BlockSpec((1, NH), fixed),
            pl.BlockSpec((NH, NH), fixed),
            pl.BlockSpec((1, NH), fixed),
        ],
        out_specs=pl.BlockSpec((bm, NH), row),
        out_shape=jax.ShapeDtypeStruct((m, NH), jnp.float32),
    )(src_raw, seq, src_t, nt, nidx, W_lin, b_lin2, freq3, phase3,
      Wq, Wk, Wv, f1w, f1b2, f2w, f2b2)


def _small_body(src_raw_ref, seq1_ref, seq2_ref, ct_ref, nt_ref, nidx_ref,
                wlin_ref, blin_ref, freq_ref, phase_ref,
                wq0_ref, wk0_ref, wv0_ref, f1w0_ref, f1b0_ref, f2w0_ref,
                f2b0_ref,
                wq1_ref, wk1_ref, wv1_ref, f1w1_ref, f1b1_ref, f2w1_ref,
                f2b1_ref,
                out_ref):
    src_conv = (jnp.dot(src_raw_ref[...], wlin_ref[...],
                        preferred_element_type=jnp.float32)
                + blin_ref[...])
    dt3 = (ct_ref[...] - nt_ref[...])[:, :, None]
    freq3, phase3 = freq_ref[...], phase_ref[...]
    nidx3 = nidx_ref[...][:, :, None]
    src_l1 = _attn_math(
        src_conv, seq1_ref[...], dt3, nidx3, freq3, phase3,
        wq0_ref[...], wk0_ref[...], wv0_ref[...],
        f1w0_ref[...], f1b0_ref[...], f2w0_ref[...], f2b0_ref[...],
        W_lin=wlin_ref[...], b_lin=blin_ref[...])
    out_ref[...] = _attn_math(
        src_l1, seq2_ref[...], dt3, nidx3, freq3, phase3,
        wq1_ref[...], wk1_ref[...], wv1_ref[...],
        f1w1_ref[...], f1b1_ref[...], f2w1_ref[...], f2b1_ref[...])


def _attn_small(src_raw, seq1, seq2, cut_t, nt, nidx,
                W_lin, b_lin2, freq3, phase3, w0, w1, bm):
    m = src_raw.shape[0]
    grid = (m // bm,)
    row = lambda i: (i, 0)
    row3 = lambda i: (i, 0, 0)
    fixed = lambda i: (0, 0)
    fixed3 = lambda i: (0, 0, 0)
    wspecs = [
        pl.BlockSpec((DM, DM), fixed),
        pl.BlockSpec((DM, DM), fixed),
        pl.BlockSpec((DM, DM), fixed),
        pl.BlockSpec((DM + NH, NH), fixed),
        pl.BlockSpec((1, NH), fixed),
        pl.BlockSpec((NH, NH), fixed),
        pl.BlockSpec((1, NH), fixed),
    ]
    return pl.pallas_call(
        _small_body,
        grid=grid,
        in_specs=[
            pl.BlockSpec((bm, NH), row),
            pl.BlockSpec((bm, KP, NH), row3),
            pl.BlockSpec((bm, KP, NH), row3),
            pl.BlockSpec((bm, 1), row),
            pl.BlockSpec((bm, KP), row),
            pl.BlockSpec((bm, KP), row),
            pl.BlockSpec((DF, NH), fixed),
            pl.BlockSpec((1, NH), fixed),
            pl.BlockSpec((1, 1, NH), fixed3),
            pl.BlockSpec((1, 1, NH), fixed3),
        ] + wspecs + wspecs,
        out_specs=pl.BlockSpec((bm, NH), row),
        out_shape=jax.ShapeDtypeStruct((m, NH), jnp.float32),
    )(src_raw, seq1, seq2, cut_t, nt, nidx, W_lin, b_lin2, freq3, phase3,
      *w0, *w1)


def kernel(node_feat, src_idx, cut_time, ngh_idx_l1, ngh_t_l1, ngh_idx_l2,
           ngh_t_l2, W_lin, b_lin, freq, phase, a0_Wq, a0_Wk, a0_Wv,
           a0_fc1_w, a0_fc1_b, a0_fc2_w, a0_fc2_b, a1_Wq, a1_Wk, a1_Wv,
           a1_fc1_w, a1_fc1_b, a1_fc2_w, a1_fc2_b):
    idx24_l2 = jnp.pad(ngh_idx_l2.astype(jnp.int32).reshape(M2, K),
                       ((0, 0), (0, KP - K)), mode="edge")
    idx24_l1 = jnp.pad(ngh_idx_l1.astype(jnp.int32),
                       ((0, 0), (0, KP - K)), mode="edge")
    nt24_l2 = jnp.pad(ngh_t_l2, ((0, 0), (0, KP - K)))
    nt24_l1 = jnp.pad(ngh_t_l1, ((0, 0), (0, KP - K)))

    n_l2p = M2 * KP
    n_l1p = B * KP
    n_l1 = M2
    n_src = B
    total = n_l2p + n_l1p + n_l1 + n_src
    pad_to = 32 * 256
    t_pad = ((total + pad_to - 1) // pad_to) * pad_to
    all_idx = jnp.concatenate([
        idx24_l2.reshape(-1),
        idx24_l1.reshape(-1),
        ngh_idx_l1.reshape(-1).astype(jnp.int32),
        src_idx.astype(jnp.int32),
        jnp.zeros((t_pad - total,), jnp.int32),
    ]).reshape(32, -1, 128)
    gathered = _sc_gather(node_feat, all_idx)
    o1 = n_l2p
    o2 = o1 + n_l1p
    o3 = o2 + n_l1
    g_l2 = gathered[:o1].reshape(M2, KP, NH)
    g_seq1 = gathered[o1:o2].reshape(B, KP, NH)
    g_l1 = gathered[o2:o3]
    g_src = gathered[o3:o3 + n_src]

    b_lin2 = b_lin.reshape(1, NH)
    freq3 = freq.reshape(1, 1, NH)
    phase3 = phase.reshape(1, 1, NH)
    f1b0 = a0_fc1_b.reshape(1, NH)
    f2b0 = a0_fc2_b.reshape(1, NH)
    f1b1 = a1_fc1_b.reshape(1, NH)
    f2b1 = a1_fc2_b.reshape(1, NH)

    ngh_l1 = _attn_big(
        g_l1, g_l2, ngh_t_l1.reshape(M2, 1), nt24_l2, idx24_l2,
        W_lin, b_lin2, freq3, phase3,
        a0_Wq, a0_Wk, a0_Wv, a0_fc1_w, f1b0, a0_fc2_w, f2b0, bm=256)

    seq2 = jnp.pad(ngh_l1.reshape(B, K, NH), ((0, 0), (0, KP - K), (0, 0)))
    w0 = (a0_Wq, a0_Wk, a0_Wv, a0_fc1_w, f1b0, a0_fc2_w, f2b0)
    w1 = (a1_Wq, a1_Wk, a1_Wv, a1_fc1_w, f1b1, a1_fc2_w, f2b1)
    out = _attn_small(
        g_src, g_seq1, seq2,
        cut_time.reshape(B, 1), nt24_l1, idx24_l1,
        W_lin, b_lin2, freq3, phase3, w0, w1, bm=128)
    return out


# speedup vs baseline: 3.0260x; 1.2204x over previous
import functools

import jax
import jax.numpy as jnp
import numpy as np
from jax import lax
from jax.experimental import pallas as pl
from jax.experimental.pallas import tpu as pltpu
from jax.experimental.pallas import tpu_sc as plsc

N, B, K, DF, NH, NHEAD = 100000, 512, 20, 128, 128, 2
DM = 2 * NH
DK = DM // NHEAD
M2 = B * K
KP = 24


def _sc_gather(table, idx):
    NW, n_ch, CH = idx.shape
    D = table.shape[1]
    per_w = n_ch * CH
    T = NW * per_w
    info = plsc.get_sparse_core_info()
    NC = info.num_cores
    mesh = plsc.VectorSubcoreMesh(core_axis_name="c", subcore_axis_name="s")
    idx_flat = idx.reshape(T)

    @functools.partial(
        pl.kernel, mesh=mesh,
        out_type=jax.ShapeDtypeStruct((T, D), jnp.float32),
        scratch_types=[
            pltpu.VMEM((CH,), jnp.int32),
            pltpu.VMEM((CH, D), jnp.float32),
            pltpu.SemaphoreType.DMA,
        ],
    )
    def k(table_hbm, idx_hbm, out_hbm, idx_v, rows_v, sem):
        wid = lax.axis_index("s") * NC + lax.axis_index("c")
        base0 = wid * per_w

        def body(c, _):
            base = base0 + c * CH
            pltpu.sync_copy(idx_hbm.at[pl.ds(base, CH)], idx_v)
            pltpu.async_copy(table_hbm.at[idx_v], rows_v, sem).wait()
            pltpu.sync_copy(rows_v, out_hbm.at[pl.ds(base, CH)])
            return _

        lax.fori_loop(0, n_ch, body, None)

    return k(table, idx_flat)


def _attn_math(src_conv, seq, dt3, nidx2, freq3, phase3,
               Wq, Wk, Wv, f1w, f1b, f2w, f2b, W_lin=None, b_lin=None):
    BM = src_conv.shape[0]
    tenc0 = jnp.cos(phase3[0])
    q = jnp.concatenate(
        [src_conv, jnp.broadcast_to(tenc0, (BM, NH))], axis=1)
    Q = jnp.dot(q, Wq, preferred_element_type=jnp.float32)
    tenc3 = jnp.cos(dt3 * freq3 + phase3)
    mask2 = nidx2 == 0
    kpad2 = lax.broadcasted_iota(jnp.int32, (BM, KP), 1) >= K
    scale = 1.0 / np.sqrt(DK)
    outs = []
    for h in range(NHEAD):
        Qh = Q[:, h * DK:(h + 1) * DK]
        Wkh = Wk[:, h * DK:(h + 1) * DK]
        Qt = lax.dot_general(Qh, Wkh, (((1,), (1,)), ((), ())),
                             preferred_element_type=jnp.float32)
        Qt_f, Qt_t = Qt[:, :NH], Qt[:, NH:]
        if W_lin is not None:
            Qr = lax.dot_general(Qt_f, W_lin, (((1,), (1,)), ((), ())),
                                 preferred_element_type=jnp.float32)
        else:
            Qr = Qt_f
        s2 = (jnp.sum(seq * Qr[:, None, :], axis=2)
              + jnp.sum(tenc3 * Qt_t[:, None, :], axis=2))
        s2 = jnp.where(mask2, -1e10, s2 * scale)
        s2 = jnp.where(kpad2, -jnp.inf, s2)
        smax = jnp.max(s2, axis=1, keepdims=True)
        e = jnp.exp(s2 - smax)
        a3 = (e / jnp.sum(e, axis=1, keepdims=True))[:, :, None]
        cr = jnp.sum(seq * a3, axis=1)
        ct = jnp.sum(tenc3 * a3, axis=1)
        Wvh = Wv[:, h * DK:(h + 1) * DK]
        if W_lin is not None:
            cr = jnp.dot(cr, W_lin,
                         preferred_element_type=jnp.float32) + b_lin
        oh = (jnp.dot(cr, Wvh[:NH, :], preferred_element_type=jnp.float32)
              + jnp.dot(ct, Wvh[NH:, :], preferred_element_type=jnp.float32))
        outs.append(oh)
    o = jnp.concatenate(outs, axis=1)
    x = jnp.concatenate([o, src_conv], axis=1)
    h1 = jax.nn.relu(jnp.dot(x, f1w, preferred_element_type=jnp.float32)
                     + f1b)
    return jnp.dot(h1, f2w, preferred_element_type=jnp.float32) + f2b


def _big_body(src_raw_ref, seq_ref, st_ref, nt_ref, nidx_ref,
              wlin_ref, blin_ref, freq_ref, phase_ref,
              wq_ref, wk_ref, wv_ref, f1w_ref, f1b_ref, f2w_ref, f2b_ref,
              out_ref):
    src_conv = (jnp.dot(src_raw_ref[...], wlin_ref[...],
                        preferred_element_type=jnp.float32)
                + blin_ref[...])
    dt3 = (st_ref[...] - nt_ref[...])[:, :, None]
    out_ref[...] = _attn_math(
        src_conv, seq_ref[...], dt3, nidx_ref[...],
        freq_ref[...], phase_ref[...],
        wq_ref[...], wk_ref[...], wv_ref[...],
        f1w_ref[...], f1b_ref[...], f2w_ref[...], f2b_ref[...],
        W_lin=wlin_ref[...], b_lin=blin_ref[...])


def _attn_big(src_raw, seq, src_t, nt, nidx, W_lin, b_lin2, freq3, phase3,
              Wq, Wk, Wv, f1w, f1b2, f2w, f2b2, bm):
    m = src_raw.shape[0]
    grid = (m // bm,)
    row = lambda i: (i, 0)
    row3 = lambda i: (i, 0, 0)
    fixed = lambda i: (0, 0)
    fixed3 = lambda i: (0, 0, 0)
    return pl.pallas_call(
        _big_body,
        grid=grid,
        in_specs=[
            pl.BlockSpec((bm, NH), row),
            pl.BlockSpec((bm, KP, NH), row3),
            pl.BlockSpec((bm, 1), row),
            pl.BlockSpec((bm, KP), row),
            pl.BlockSpec((bm, KP), row),
            pl.BlockSpec((DF, NH), fixed),
            pl.BlockSpec((1, NH), fixed),
            pl.BlockSpec((1, 1, NH), fixed3),
            pl.BlockSpec((1, 1, NH), fixed3),
            pl.BlockSpec((DM, DM), fixed),
            pl.BlockSpec((DM, DM), fixed),
            pl.BlockSpec((DM, DM), fixed),
            pl.BlockSpec((DM + NH, NH), fixed),
            pl.BlockSpec((1, NH), fixed),
            pl.BlockSpec((NH, NH), fixed),
            pl.BlockSpec((1, NH), fixed),
        ],
        out_specs=pl.BlockSpec((bm, NH), row),
        out_shape=jax.ShapeDtypeStruct((m, NH), jnp.float32),
    )(src_raw, seq, src_t, nt, nidx, W_lin, b_lin2, freq3, phase3,
      Wq, Wk, Wv, f1w, f1b2, f2w, f2b2)


def _small_body(src_raw_ref, seq1_ref, seq2_ref, ct_ref, nt_ref, nidx_ref,
                wlin_ref, blin_ref, freq_ref, phase_ref,
                wq0_ref, wk0_ref, wv0_ref, f1w0_ref, f1b0_ref, f2w0_ref,
                f2b0_ref,
                wq1_ref, wk1_ref, wv1_ref, f1w1_ref, f1b1_ref, f2w1_ref,
                f2b1_ref,
                out_ref):
    src_conv = (jnp.dot(src_raw_ref[...], wlin_ref[...],
                        preferred_element_type=jnp.float32)
                + blin_ref[...])
    dt3 = (ct_ref[...] - nt_ref[...])[:, :, None]
    freq3, phase3 = freq_ref[...], phase_ref[...]
    nidx2 = nidx_ref[...]
    src_l1 = _attn_math(
        src_conv, seq1_ref[...], dt3, nidx2, freq3, phase3,
        wq0_ref[...], wk0_ref[...], wv0_ref[...],
        f1w0_ref[...], f1b0_ref[...], f2w0_ref[...], f2b0_ref[...],
        W_lin=wlin_ref[...], b_lin=blin_ref[...])
    out_ref[...] = _attn_math(
        src_l1, seq2_ref[...], dt3, nidx2, freq3, phase3,
        wq1_ref[...], wk1_ref[...], wv1_ref[...],
        f1w1_ref[...], f1b1_ref[...], f2w1_ref[...], f2b1_ref[...])


def _attn_small(src_raw, seq1, seq2, cut_t, nt, nidx,
                W_lin, b_lin2, freq3, phase3, w0, w1, bm):
    m = src_raw.shape[0]
    grid = (m // bm,)
    row = lambda i: (i, 0)
    row3 = lambda i: (i, 0, 0)
    fixed = lambda i: (0, 0)
    fixed3 = lambda i: (0, 0, 0)
    wspecs = [
        pl.BlockSpec((DM, DM), fixed),
        pl.BlockSpec((DM, DM), fixed),
        pl.BlockSpec((DM, DM), fixed),
        pl.BlockSpec((DM + NH, NH), fixed),
        pl.BlockSpec((1, NH), fixed),
        pl.BlockSpec((NH, NH), fixed),
        pl.BlockSpec((1, NH), fixed),
    ]
    return pl.pallas_call(
        _small_body,
        grid=grid,
        in_specs=[
            pl.BlockSpec((bm, NH), row),
            pl.BlockSpec((bm, KP, NH), row3),
            pl.BlockSpec((bm, KP, NH), row3),
            pl.BlockSpec((bm, 1), row),
            pl.BlockSpec((bm, KP), row),
            pl.BlockSpec((bm, KP), row),
            pl.BlockSpec((DF, NH), fixed),
            pl.BlockSpec((1, NH), fixed),
            pl.BlockSpec((1, 1, NH), fixed3),
            pl.BlockSpec((1, 1, NH), fixed3),
        ] + wspecs + wspecs,
        out_specs=pl.BlockSpec((bm, NH), row),
        out_shape=jax.ShapeDtypeStruct((m, NH), jnp.float32),
    )(src_raw, seq1, seq2, cut_t, nt, nidx, W_lin, b_lin2, freq3, phase3,
      *w0, *w1)


def kernel(node_feat, src_idx, cut_time, ngh_idx_l1, ngh_t_l1, ngh_idx_l2,
           ngh_t_l2, W_lin, b_lin, freq, phase, a0_Wq, a0_Wk, a0_Wv,
           a0_fc1_w, a0_fc1_b, a0_fc2_w, a0_fc2_b, a1_Wq, a1_Wk, a1_Wv,
           a1_fc1_w, a1_fc1_b, a1_fc2_w, a1_fc2_b):
    idx24_l2 = jnp.pad(ngh_idx_l2.astype(jnp.int32).reshape(M2, K),
                       ((0, 0), (0, KP - K)), mode="edge")
    idx24_l1 = jnp.pad(ngh_idx_l1.astype(jnp.int32),
                       ((0, 0), (0, KP - K)), mode="edge")
    nt24_l2 = jnp.pad(ngh_t_l2, ((0, 0), (0, KP - K)))
    nt24_l1 = jnp.pad(ngh_t_l1, ((0, 0), (0, KP - K)))

    MH = M2 // 2
    idx_l1_flat = ngh_idx_l1.reshape(-1).astype(jnp.int32)
    idx_a = jnp.concatenate([
        idx24_l2[:MH].reshape(-1),
        idx_l1_flat,
        src_idx.astype(jnp.int32),
        idx_l1_flat[:1536],
    ]).reshape(32, -1, 128)
    idx_b = jnp.concatenate([
        idx24_l2[MH:].reshape(-1),
        idx24_l1.reshape(-1),
    ]).reshape(32, -1, 128)
    ga = _sc_gather(node_feat, idx_a)
    gb = _sc_gather(node_feat, idx_b)
    g_l2a = ga[:MH * KP].reshape(MH, KP, NH)
    g_l1 = ga[MH * KP:MH * KP + M2]
    g_src = ga[MH * KP + M2:MH * KP + M2 + B]
    g_l2b = gb[:MH * KP].reshape(MH, KP, NH)
    g_seq1 = gb[MH * KP:].reshape(B, KP, NH)

    b_lin2 = b_lin.reshape(1, NH)
    freq3 = freq.reshape(1, 1, NH)
    phase3 = phase.reshape(1, 1, NH)
    f1b0 = a0_fc1_b.reshape(1, NH)
    f2b0 = a0_fc2_b.reshape(1, NH)
    f1b1 = a1_fc1_b.reshape(1, NH)
    f2b1 = a1_fc2_b.reshape(1, NH)

    st2 = ngh_t_l1.reshape(M2, 1)
    ngh_l1_a = _attn_big(
        g_l1[:MH], g_l2a, st2[:MH], nt24_l2[:MH], idx24_l2[:MH],
        W_lin, b_lin2, freq3, phase3,
        a0_Wq, a0_Wk, a0_Wv, a0_fc1_w, f1b0, a0_fc2_w, f2b0, bm=256)
    ngh_l1_b = _attn_big(
        g_l1[MH:], g_l2b, st2[MH:], nt24_l2[MH:], idx24_l2[MH:],
        W_lin, b_lin2, freq3, phase3,
        a0_Wq, a0_Wk, a0_Wv, a0_fc1_w, f1b0, a0_fc2_w, f2b0, bm=256)
    ngh_l1 = jnp.concatenate([ngh_l1_a, ngh_l1_b], axis=0)

    seq2 = jnp.pad(ngh_l1.reshape(B, K, NH), ((0, 0), (0, KP - K), (0, 0)))
    w0 = (a0_Wq, a0_Wk, a0_Wv, a0_fc1_w, f1b0, a0_fc2_w, f2b0)
    w1 = (a1_Wq, a1_Wk, a1_Wv, a1_fc1_w, f1b1, a1_fc2_w, f2b1)
    out = _attn_small(
        g_src, g_seq1, seq2,
        cut_time.reshape(B, 1), nt24_l1, idx24_l1,
        W_lin, b_lin2, freq3, phase3, w0, w1, bm=128)
    return out


# speedup vs baseline: 4.8569x; 1.6051x over previous
import functools

import jax
import jax.numpy as jnp
import numpy as np
from jax import lax
from jax.experimental import pallas as pl
from jax.experimental.pallas import tpu as pltpu
from jax.experimental.pallas import tpu_sc as plsc

N, B, K, DF, NH, NHEAD = 100000, 512, 20, 128, 128, 2
DM = 2 * NH
DK = DM // NHEAD
M2 = B * K
KP = 24


def _sc_gather(table, idx):
    NW, n_ch, CH = idx.shape
    D = table.shape[1]
    per_w = n_ch * CH
    T = NW * per_w
    info = plsc.get_sparse_core_info()
    NC = info.num_cores
    mesh = plsc.VectorSubcoreMesh(core_axis_name="c", subcore_axis_name="s")
    idx_flat = idx.reshape(T)

    @functools.partial(
        pl.kernel, mesh=mesh,
        out_type=jax.ShapeDtypeStruct((T, D), jnp.float32),
        scratch_types=[
            pltpu.VMEM((CH,), jnp.int32),
            pltpu.VMEM((CH, D), jnp.float32),
            pltpu.SemaphoreType.DMA,
        ],
    )
    def k(table_hbm, idx_hbm, out_hbm, idx_v, rows_v, sem):
        wid = lax.axis_index("s") * NC + lax.axis_index("c")
        base0 = wid * per_w

        def body(c, _):
            base = base0 + c * CH
            pltpu.sync_copy(idx_hbm.at[pl.ds(base, CH)], idx_v)
            pltpu.async_copy(table_hbm.at[idx_v], rows_v, sem).wait()
            pltpu.sync_copy(rows_v, out_hbm.at[pl.ds(base, CH)])
            return _

        lax.fori_loop(0, n_ch, body, None)

    return k(table, idx_flat)


def _fast_cos(x):
    t = x * (1.0 / (2.0 * np.pi))
    r = t - jnp.round(t)
    u = r * r
    poly = (3.1415925801461613
            + u * (-5.16770688438281
                   + u * (2.550031461491303
                          + u * (-0.5980456619394252
                                 + u * 0.07722107147153531))))
    s = r * poly
    return 1.0 - 2.0 * s * s


def _attn_math(src_conv, seq, dt3, nidx2, freq3, phase3,
               Wq, Wk, Wv, f1w, f1b, f2w, f2b, W_lin=None, b_lin=None):
    BM = src_conv.shape[0]
    tenc0 = jnp.cos(phase3[0])
    q = jnp.concatenate(
        [src_conv, jnp.broadcast_to(tenc0, (BM, NH))], axis=1)
    Q = jnp.dot(q, Wq, preferred_element_type=jnp.float32)
    tenc3 = _fast_cos(dt3 * freq3 + phase3)
    mask2 = nidx2 == 0
    kpad2 = lax.broadcasted_iota(jnp.int32, (BM, KP), 1) >= K
    scale = 1.0 / np.sqrt(DK)
    outs = []
    for h in range(NHEAD):
        Qh = Q[:, h * DK:(h + 1) * DK]
        Wkh = Wk[:, h * DK:(h + 1) * DK]
        Qt = lax.dot_general(Qh, Wkh, (((1,), (1,)), ((), ())),
                             preferred_element_type=jnp.float32)
        Qt_f, Qt_t = Qt[:, :NH], Qt[:, NH:]
        if W_lin is not None:
            Qr = lax.dot_general(Qt_f, W_lin, (((1,), (1,)), ((), ())),
                                 preferred_element_type=jnp.float32)
        else:
            Qr = Qt_f
        s2 = (jnp.sum(seq * Qr[:, None, :], axis=2)
              + jnp.sum(tenc3 * Qt_t[:, None, :], axis=2))
        s2 = jnp.where(mask2, -1e10, s2 * scale)
        s2 = jnp.where(kpad2, -jnp.inf, s2)
        smax = jnp.max(s2, axis=1, keepdims=True)
        e = jnp.exp(s2 - smax)
        a3 = (e / jnp.sum(e, axis=1, keepdims=True))[:, :, None]
        cr = jnp.sum(seq * a3, axis=1)
        ct = jnp.sum(tenc3 * a3, axis=1)
        Wvh = Wv[:, h * DK:(h + 1) * DK]
        if W_lin is not None:
            cr = jnp.dot(cr, W_lin,
                         preferred_element_type=jnp.float32) + b_lin
        oh = (jnp.dot(cr, Wvh[:NH, :], preferred_element_type=jnp.float32)
              + jnp.dot(ct, Wvh[NH:, :], preferred_element_type=jnp.float32))
        outs.append(oh)
    o = jnp.concatenate(outs, axis=1)
    x = jnp.concatenate([o, src_conv], axis=1)
    h1 = jax.nn.relu(jnp.dot(x, f1w, preferred_element_type=jnp.float32)
                     + f1b)
    return jnp.dot(h1, f2w, preferred_element_type=jnp.float32) + f2b


def _big_body(src_raw_ref, seq_ref, st_ref, nt_ref, nidx_ref,
              wlin_ref, blin_ref, freq_ref, phase_ref,
              wq_ref, wk_ref, wv_ref, f1w_ref, f1b_ref, f2w_ref, f2b_ref,
              out_ref):
    src_conv = (jnp.dot(src_raw_ref[...], wlin_ref[...],
                        preferred_element_type=jnp.float32)
                + blin_ref[...])
    dt3 = (st_ref[...] - nt_ref[...])[:, :, None]
    out_ref[...] = _attn_math(
        src_conv, seq_ref[...], dt3, nidx_ref[...],
        freq_ref[...], phase_ref[...],
        wq_ref[...], wk_ref[...], wv_ref[...],
        f1w_ref[...], f1b_ref[...], f2w_ref[...], f2b_ref[...],
        W_lin=wlin_ref[...], b_lin=blin_ref[...])


def _attn_big(src_raw, seq, src_t, nt, nidx, W_lin, b_lin2, freq3, phase3,
              Wq, Wk, Wv, f1w, f1b2, f2w, f2b2, bm):
    m = src_raw.shape[0]
    grid = (m // bm,)
    row = lambda i: (i, 0)
    row3 = lambda i: (i, 0, 0)
    fixed = lambda i: (0, 0)
    fixed3 = lambda i: (0, 0, 0)
    return pl.pallas_call(
        _big_body,
        grid=grid,
        in_specs=[
            pl.BlockSpec((bm, NH), row),
            pl.BlockSpec((bm, KP, NH), row3),
            pl.BlockSpec((bm, 1), row),
            pl.BlockSpec((bm, KP), row),
            pl.BlockSpec((bm, KP), row),
            pl.BlockSpec((DF, NH), fixed),
            pl.BlockSpec((1, NH), fixed),
            pl.BlockSpec((1, 1, NH), fixed3),
            pl.BlockSpec((1, 1, NH), fixed3),
            pl.BlockSpec((DM, DM), fixed),
            pl.BlockSpec((DM, DM), fixed),
            pl.BlockSpec((DM, DM), fixed),
            pl.BlockSpec((DM + NH, NH), fixed),
            pl.BlockSpec((1, NH), fixed),
            pl.BlockSpec((NH, NH), fixed),
            pl.BlockSpec((1, NH), fixed),
        ],
        out_specs=pl.BlockSpec((bm, NH), row),
        out_shape=jax.ShapeDtypeStruct((m, NH), jnp.float32),
    )(src_raw, seq, src_t, nt, nidx, W_lin, b_lin2, freq3, phase3,
      Wq, Wk, Wv, f1w, f1b2, f2w, f2b2)


def _small_body(src_raw_ref, seq1_ref, seq2_ref, ct_ref, nt_ref, nidx_ref,
                wlin_ref, blin_ref, freq_ref, phase_ref,
                wq0_ref, wk0_ref, wv0_ref, f1w0_ref, f1b0_ref, f2w0_ref,
                f2b0_ref,
                wq1_ref, wk1_ref, wv1_ref, f1w1_ref, f1b1_ref, f2w1_ref,
                f2b1_ref,
                out_ref):
    src_conv = (jnp.dot(src_raw_ref[...], wlin_ref[...],
                        preferred_element_type=jnp.float32)
                + blin_ref[...])
    dt3 = (ct_ref[...] - nt_ref[...])[:, :, None]
    freq3, phase3 = freq_ref[...], phase_ref[...]
    nidx2 = nidx_ref[...]
    src_l1 = _attn_math(
        src_conv, seq1_ref[...], dt3, nidx2, freq3, phase3,
        wq0_ref[...], wk0_ref[...], wv0_ref[...],
        f1w0_ref[...], f1b0_ref[...], f2w0_ref[...], f2b0_ref[...],
        W_lin=wlin_ref[...], b_lin=blin_ref[...])
    out_ref[...] = _attn_math(
        src_l1, seq2_ref[...], dt3, nidx2, freq3, phase3,
        wq1_ref[...], wk1_ref[...], wv1_ref[...],
        f1w1_ref[...], f1b1_ref[...], f2w1_ref[...], f2b1_ref[...])


def _attn_small(src_raw, seq1, seq2, cut_t, nt, nidx,
                W_lin, b_lin2, freq3, phase3, w0, w1, bm):
    m = src_raw.shape[0]
    grid = (m // bm,)
    row = lambda i: (i, 0)
    row3 = lambda i: (i, 0, 0)
    fixed = lambda i: (0, 0)
    fixed3 = lambda i: (0, 0, 0)
    wspecs = [
        pl.BlockSpec((DM, DM), fixed),
        pl.BlockSpec((DM, DM), fixed),
        pl.BlockSpec((DM, DM), fixed),
        pl.BlockSpec((DM + NH, NH), fixed),
        pl.BlockSpec((1, NH), fixed),
        pl.BlockSpec((NH, NH), fixed),
        pl.BlockSpec((1, NH), fixed),
    ]
    return pl.pallas_call(
        _small_body,
        grid=grid,
        in_specs=[
            pl.BlockSpec((bm, NH), row),
            pl.BlockSpec((bm, KP, NH), row3),
            pl.BlockSpec((bm, KP, NH), row3),
            pl.BlockSpec((bm, 1), row),
            pl.BlockSpec((bm, KP), row),
            pl.BlockSpec((bm, KP), row),
            pl.BlockSpec((DF, NH), fixed),
            pl.BlockSpec((1, NH), fixed),
            pl.BlockSpec((1, 1, NH), fixed3),
            pl.BlockSpec((1, 1, NH), fixed3),
        ] + wspecs + wspecs,
        out_specs=pl.BlockSpec((bm, NH), row),
        out_shape=jax.ShapeDtypeStruct((m, NH), jnp.float32),
    )(src_raw, seq1, seq2, cut_t, nt, nidx, W_lin, b_lin2, freq3, phase3,
      *w0, *w1)


def kernel(node_feat, src_idx, cut_time, ngh_idx_l1, ngh_t_l1, ngh_idx_l2,
           ngh_t_l2, W_lin, b_lin, freq, phase, a0_Wq, a0_Wk, a0_Wv,
           a0_fc1_w, a0_fc1_b, a0_fc2_w, a0_fc2_b, a1_Wq, a1_Wk, a1_Wv,
           a1_fc1_w, a1_fc1_b, a1_fc2_w, a1_fc2_b):
    idx24_l2 = jnp.pad(ngh_idx_l2.astype(jnp.int32).reshape(M2, K),
                       ((0, 0), (0, KP - K)), mode="edge")
    idx24_l1 = jnp.pad(ngh_idx_l1.astype(jnp.int32),
                       ((0, 0), (0, KP - K)), mode="edge")
    nt24_l2 = jnp.pad(ngh_t_l2, ((0, 0), (0, KP - K)))
    nt24_l1 = jnp.pad(ngh_t_l1, ((0, 0), (0, KP - K)))

    MH = M2 // 2
    idx_l1_flat = ngh_idx_l1.reshape(-1).astype(jnp.int32)
    idx_a = jnp.concatenate([
        idx24_l2[:MH].reshape(-1),
        idx_l1_flat,
        src_idx.astype(jnp.int32),
        idx_l1_flat[:1536],
    ]).reshape(32, -1, 128)
    idx_b = jnp.concatenate([
        idx24_l2[MH:].reshape(-1),
        idx24_l1.reshape(-1),
    ]).reshape(32, -1, 128)
    ga = _sc_gather(node_feat, idx_a)
    gb = _sc_gather(node_feat, idx_b)
    g_l2a = ga[:MH * KP].reshape(MH, KP, NH)
    g_l1 = ga[MH * KP:MH * KP + M2]
    g_src = ga[MH * KP + M2:MH * KP + M2 + B]
    g_l2b = gb[:MH * KP].reshape(MH, KP, NH)
    g_seq1 = gb[MH * KP:].reshape(B, KP, NH)

    b_lin2 = b_lin.reshape(1, NH)
    freq3 = freq.reshape(1, 1, NH)
    phase3 = phase.reshape(1, 1, NH)
    f1b0 = a0_fc1_b.reshape(1, NH)
    f2b0 = a0_fc2_b.reshape(1, NH)
    f1b1 = a1_fc1_b.reshape(1, NH)
    f2b1 = a1_fc2_b.reshape(1, NH)

    st2 = ngh_t_l1.reshape(M2, 1)
    ngh_l1_a = _attn_big(
        g_l1[:MH], g_l2a, st2[:MH], nt24_l2[:MH], idx24_l2[:MH],
        W_lin, b_lin2, freq3, phase3,
        a0_Wq, a0_Wk, a0_Wv, a0_fc1_w, f1b0, a0_fc2_w, f2b0, bm=256)
    ngh_l1_b = _attn_big(
        g_l1[MH:], g_l2b, st2[MH:], nt24_l2[MH:], idx24_l2[MH:],
        W_lin, b_lin2, freq3, phase3,
        a0_Wq, a0_Wk, a0_Wv, a0_fc1_w, f1b0, a0_fc2_w, f2b0, bm=256)
    ngh_l1 = jnp.concatenate([ngh_l1_a, ngh_l1_b], axis=0)

    seq2 = jnp.pad(ngh_l1.reshape(B, K, NH), ((0, 0), (0, KP - K), (0, 0)))
    w0 = (a0_Wq, a0_Wk, a0_Wv, a0_fc1_w, f1b0, a0_fc2_w, f2b0)
    w1 = (a1_Wq, a1_Wk, a1_Wv, a1_fc1_w, f1b1, a1_fc2_w, f2b1)
    out = _attn_small(
        g_src, g_seq1, seq2,
        cut_time.reshape(B, 1), nt24_l1, idx24_l1,
        W_lin, b_lin2, freq3, phase3, w0, w1, bm=128)
    return out
